# baseline (speedup 1.0000x reference)
import functools

import jax
import jax.numpy as jnp
from jax import lax
from jax.experimental import pallas as pl
from jax.experimental.pallas import tpu as pltpu
from jax.experimental.pallas import tpu_sc as plsc

SEQ = 8192
DIM = 1024

_NUM_CORES = 2
_ROWS_PER_C = SEQ // _NUM_CORES
_CHUNK = 512
_NBUF = 3
_NCHUNK = _ROWS_PER_C // _CHUNK

_mesh = plsc.ScalarSubcoreMesh(axis_name="c", num_cores=_NUM_CORES)


@functools.partial(
    pl.kernel,
    mesh=_mesh,
    out_type=jax.ShapeDtypeStruct((SEQ, 1, DIM), jnp.float32),
    scratch_types=(
        [pltpu.VMEM_SHARED((_CHUNK, 1, DIM), jnp.float32) for _ in range(_NBUF)]
        + [pltpu.SemaphoreType.DMA for _ in range(2 * _NBUF)]
    ),
)
def _sc_copy(embed_hbm, out_hbm, *scratch):
    bufs = scratch[:_NBUF]
    isems = scratch[_NBUF:2 * _NBUF]
    osems = scratch[2 * _NBUF:]
    base = lax.axis_index("c") * _ROWS_PER_C

    def in_copy(i):
        return pltpu.async_copy(
            embed_hbm.at[pl.ds(base + i * _CHUNK, _CHUNK)],
            bufs[i % _NBUF].at[:, 0, :],
            isems[i % _NBUF],
        )

    def out_copy(i):
        return pltpu.async_copy(
            bufs[i % _NBUF],
            out_hbm.at[pl.ds(base + i * _CHUNK, _CHUNK)],
            osems[i % _NBUF],
        )

    ins = [None] * _NCHUNK
    outs = [None] * _NCHUNK
    for i in range(min(_NBUF, _NCHUNK)):
        ins[i] = in_copy(i)
    for i in range(_NCHUNK):
        ins[i].wait()
        outs[i] = out_copy(i)
        nxt = i + _NBUF
        if nxt < _NCHUNK:
            outs[i].wait()
            ins[nxt] = in_copy(nxt)
    for i in range(max(0, _NCHUNK - _NBUF), _NCHUNK):
        outs[i].wait()


def kernel(input, embed):
    return _sc_copy(embed)


# speedup vs baseline: 1.0758x; 1.0758x over previous
import functools

import jax
import jax.numpy as jnp
from jax import lax
from jax.experimental import pallas as pl
from jax.experimental.pallas import tpu as pltpu
from jax.experimental.pallas import tpu_sc as plsc

SEQ = 8192
DIM = 1024

_NUM_CORES = 2
_NUM_SUBCORES = 16
_NW = _NUM_CORES * _NUM_SUBCORES
_ROWS_PER_W = SEQ // _NW
_CHUNK = 32
_NBUF = 3
_NCHUNK = _ROWS_PER_W // _CHUNK

_mesh = plsc.VectorSubcoreMesh(core_axis_name="c", subcore_axis_name="s")


@functools.partial(
    pl.kernel,
    mesh=_mesh,
    out_type=jax.ShapeDtypeStruct((SEQ, 1, DIM), jnp.float32),
    scratch_types=(
        [pltpu.VMEM((_CHUNK, 1, DIM), jnp.float32) for _ in range(_NBUF)]
        + [pltpu.SemaphoreType.DMA for _ in range(2 * _NBUF)]
    ),
)
def _sc_copy(embed_hbm, out_hbm, *scratch):
    bufs = scratch[:_NBUF]
    isems = scratch[_NBUF:2 * _NBUF]
    osems = scratch[2 * _NBUF:]
    wid = lax.axis_index("s") * _NUM_CORES + lax.axis_index("c")
    base = wid * _ROWS_PER_W

    def in_copy(i):
        return pltpu.async_copy(
            embed_hbm.at[pl.ds(base + i * _CHUNK, _CHUNK)],
            bufs[i % _NBUF].at[:, 0, :],
            isems[i % _NBUF],
        )

    def out_copy(i):
        return pltpu.async_copy(
            bufs[i % _NBUF],
            out_hbm.at[pl.ds(base + i * _CHUNK, _CHUNK)],
            osems[i % _NBUF],
        )

    ins = [None] * _NCHUNK
    outs = [None] * _NCHUNK
    for i in range(min(_NBUF, _NCHUNK)):
        ins[i] = in_copy(i)
    for i in range(_NCHUNK):
        ins[i].wait()
        outs[i] = out_copy(i)
        nxt = i + _NBUF
        if nxt < _NCHUNK:
            outs[i].wait()
            ins[nxt] = in_copy(nxt)
    for i in range(max(0, _NCHUNK - _NBUF), _NCHUNK):
        outs[i].wait()


def kernel(input, embed):
    return _sc_copy(embed)
